# baseline (speedup 1.0000x reference)
import functools

import jax
import jax.numpy as jnp
import numpy as np
from jax.experimental import pallas as pl
from jax.experimental.pallas import tpu as pltpu

_ROT = ((13, 15, 26, 6), (17, 29, 16, 24))
_KS0 = np.uint32(0)
_KS1 = np.uint32(42)
_KS2 = np.uint32(_KS0 ^ _KS1 ^ np.uint32(0x1BD11BDA))
_KS = (_KS0, _KS1, _KS2)
_TINY = np.float32(np.finfo(np.float32).tiny)
_SPAN = np.float32(np.float32(1.0) - _TINY)
_BIG_IDX = np.int32(2**30)


def _rotl(x, d):
    return (x << np.uint32(d)) | (x >> np.uint32(32 - d))


def _threefry_bits(x1):
    x0 = x1
    x1 = _rotl(x1, _ROT[0][0]) ^ x0
    for r in _ROT[0][1:]:
        x0 = x0 + x1
        x1 = _rotl(x1, r)
        x1 = x1 ^ x0
    x0 = x0 + _KS[1]
    x1 = x1 + np.uint32(_KS[2] + np.uint32(1))
    for i in range(1, 5):
        for r in _ROT[i % 2]:
            x0 = x0 + x1
            x1 = _rotl(x1, r)
            x1 = x1 ^ x0
        x0 = x0 + _KS[(i + 1) % 3]
        x1 = x1 + np.uint32(_KS[(i + 2) % 3] + np.uint32(i + 1))
    return x0 ^ x1


def _sample_kernel(base_ref, logits_ref, out_ref, rv_ref, ri_ref, *, ncols,
                   width, chunk, rows_per_blk, nblk, rem):
    r = pl.program_id(0)
    k = pl.program_id(1)

    @pl.when(k == 0)
    def _init():
        rv_ref[...] = jnp.full_like(rv_ref, -jnp.inf)
        ri_ref[...] = jnp.full_like(ri_ref, _BIG_IDX)

    def do_chunk(s, masked):
        sl = pl.ds(s * chunk, chunk)
        ctr = base_ref[:, sl] + (r * (rows_per_blk * ncols) + k * width)
        bits = _threefry_bits(ctr.astype(jnp.uint32))
        fb = (bits >> np.uint32(9)) | np.uint32(0x3F800000)
        f = jax.lax.bitcast_convert_type(fb, jnp.float32) - np.float32(1.0)
        u = jnp.maximum(_TINY, f * _SPAN + _TINY)
        g = -jnp.log(-jnp.log(u))
        val = g + logits_ref[:, sl]
        if masked:
            lane = (jax.lax.broadcasted_iota(jnp.int32, (rows_per_blk, chunk), 1)
                    + s * chunk)
            val = jnp.where(lane < rem, val, -jnp.inf)
        upd = val > rv_ref[:, sl]
        rv_ref[:, sl] = jnp.where(upd, val, rv_ref[:, sl])
        ri_ref[:, sl] = jnp.where(upd, ctr, ri_ref[:, sl])

    nchunk = width // chunk

    @pl.when(k < nblk - 1)
    def _steady():
        for s in range(nchunk):
            do_chunk(s, masked=False)

    @pl.when(k == nblk - 1)
    def _finish():
        for s in range(nchunk):
            do_chunk(s, masked=True)
        rv = rv_ref[...]
        ri = ri_ref[...]
        m = jnp.max(rv, axis=1, keepdims=True)
        cand = jnp.where(rv == m, ri, _BIG_IDX)
        a = jnp.min(cand, axis=1, keepdims=True)
        rowbase = (jax.lax.broadcasted_iota(jnp.int32, (rows_per_blk, 1), 0)
                   + r * rows_per_blk) * ncols + np.int32(42)
        out_ref[...] = a - rowbase


@jax.jit
def kernel(logits):
    nrows, ncols = logits.shape
    rows_per_blk = nrows // 2
    width = 49152
    nblk = pl.cdiv(ncols, width)
    rem = ncols - (nblk - 1) * width

    base = (jax.lax.broadcasted_iota(jnp.int32, (rows_per_blk, width), 0)
            * ncols
            + jax.lax.broadcasted_iota(jnp.int32, (rows_per_blk, width), 1)
            + 42)

    out = pl.pallas_call(
        functools.partial(_sample_kernel, ncols=ncols, width=width,
                          chunk=512, rows_per_blk=rows_per_blk, nblk=nblk,
                          rem=rem),
        grid=(2, nblk),
        in_specs=[
            pl.BlockSpec((rows_per_blk, width), lambda r, k: (0, 0)),
            pl.BlockSpec((rows_per_blk, width), lambda r, k: (r, k)),
        ],
        out_specs=pl.BlockSpec((rows_per_blk, 1), lambda r, k: (r, 0)),
        out_shape=jax.ShapeDtypeStruct((nrows, 1), jnp.int32),
        scratch_shapes=[
            pltpu.VMEM((rows_per_blk, width), jnp.float32),
            pltpu.VMEM((rows_per_blk, width), jnp.int32),
        ],
        compiler_params=pltpu.CompilerParams(
            dimension_semantics=("parallel", "arbitrary"),
        ),
    )(base, logits)
    return out.reshape(nrows).astype(jnp.int64)


# speedup vs baseline: 1.0017x; 1.0017x over previous
import functools

import jax
import jax.numpy as jnp
import numpy as np
from jax.experimental import pallas as pl
from jax.experimental.pallas import tpu as pltpu

_ROT = ((13, 15, 26, 6), (17, 29, 16, 24))
_KS0 = np.uint32(0)
_KS1 = np.uint32(42)
_KS2 = np.uint32(_KS0 ^ _KS1 ^ np.uint32(0x1BD11BDA))
_KS = (_KS0, _KS1, _KS2)
_TINY = np.float32(np.finfo(np.float32).tiny)
_SPAN = np.float32(np.float32(1.0) - _TINY)
_BIG_IDX = np.int32(2**30)


def _rotl(x, d):
    return (x << np.uint32(d)) | (x >> np.uint32(32 - d))


def _threefry_bits(x1):
    x0 = x1
    x1 = _rotl(x1, _ROT[0][0]) ^ x0
    for r in _ROT[0][1:]:
        x0 = x0 + x1
        x1 = _rotl(x1, r)
        x1 = x1 ^ x0
    x0 = x0 + _KS[1]
    x1 = x1 + np.uint32(_KS[2] + np.uint32(1))
    for i in range(1, 5):
        for r in _ROT[i % 2]:
            x0 = x0 + x1
            x1 = _rotl(x1, r)
            x1 = x1 ^ x0
        x0 = x0 + _KS[(i + 1) % 3]
        x1 = x1 + np.uint32(_KS[(i + 2) % 3] + np.uint32(i + 1))
    return x0 ^ x1


def _sample_kernel(base_ref, logits_ref, out_ref, rv_ref, ri_ref, *, ncols,
                   width, chunk, rows_per_blk, nblk, rem):
    r = pl.program_id(0)
    k = pl.program_id(1)

    @pl.when(k == 0)
    def _init():
        rv_ref[...] = jnp.full_like(rv_ref, -jnp.inf)
        ri_ref[...] = jnp.full_like(ri_ref, _BIG_IDX)

    def do_chunk(s, masked):
        sl = pl.ds(s * chunk, chunk)
        ctr = base_ref[:, sl] + (r * (rows_per_blk * ncols) + k * width)
        bits = _threefry_bits(ctr.astype(jnp.uint32))
        fb = (bits >> np.uint32(9)) | np.uint32(0x3F800000)
        f = jax.lax.bitcast_convert_type(fb, jnp.float32) - np.float32(1.0)
        u = jnp.maximum(_TINY, f * _SPAN + _TINY)
        g = -jnp.log(-jnp.log(u))
        val = g + logits_ref[:, sl]
        if masked:
            lane = (jax.lax.broadcasted_iota(jnp.int32, (rows_per_blk, chunk), 1)
                    + s * chunk)
            val = jnp.where(lane < rem, val, -jnp.inf)
        upd = val > rv_ref[:, sl]
        rv_ref[:, sl] = jnp.where(upd, val, rv_ref[:, sl])
        ri_ref[:, sl] = jnp.where(upd, ctr, ri_ref[:, sl])

    nchunk = width // chunk

    @pl.when(k < nblk - 1)
    def _steady():
        for s in range(nchunk):
            do_chunk(s, masked=False)

    @pl.when(k == nblk - 1)
    def _finish():
        for s in range(nchunk):
            do_chunk(s, masked=True)
        rv = rv_ref[...]
        ri = ri_ref[...]
        m = jnp.max(rv, axis=1, keepdims=True)
        cand = jnp.where(rv == m, ri, _BIG_IDX)
        a = jnp.min(cand, axis=1, keepdims=True)
        rowbase = (jax.lax.broadcasted_iota(jnp.int32, (rows_per_blk, 1), 0)
                   + r * rows_per_blk) * ncols + np.int32(42)
        out_ref[...] = a - rowbase


@jax.jit
def kernel(logits):
    nrows, ncols = logits.shape
    rows_per_blk = nrows // 2
    width = 32768
    nblk = pl.cdiv(ncols, width)
    rem = ncols - (nblk - 1) * width

    base = (jax.lax.broadcasted_iota(jnp.int32, (rows_per_blk, width), 0)
            * ncols
            + jax.lax.broadcasted_iota(jnp.int32, (rows_per_blk, width), 1)
            + 42)

    out = pl.pallas_call(
        functools.partial(_sample_kernel, ncols=ncols, width=width,
                          chunk=512, rows_per_blk=rows_per_blk, nblk=nblk,
                          rem=rem),
        grid=(2, nblk),
        in_specs=[
            pl.BlockSpec((rows_per_blk, width), lambda r, k: (0, 0)),
            pl.BlockSpec((rows_per_blk, width), lambda r, k: (r, k)),
        ],
        out_specs=pl.BlockSpec((rows_per_blk, 1), lambda r, k: (r, 0)),
        out_shape=jax.ShapeDtypeStruct((nrows, 1), jnp.int32),
        scratch_shapes=[
            pltpu.VMEM((rows_per_blk, width), jnp.float32),
            pltpu.VMEM((rows_per_blk, width), jnp.int32),
        ],
        compiler_params=pltpu.CompilerParams(
            dimension_semantics=("parallel", "arbitrary"),
        ),
    )(base, logits)
    return out.reshape(nrows).astype(jnp.int64)
